# Initial kernel scaffold; baseline (speedup 1.0000x reference)
import functools

import jax
import jax.numpy as jnp
from jax import lax
from jax.experimental import pallas as pl
from jax.experimental.pallas import tpu as pltpu
from jax.experimental.pallas import tpu_sc as plsc

N = 10000
E = 320000
D = 128

NC = 2
NS = 16
NW = NC * NS
L = 16

CHUNK = 128
EPT = E // NW
N_CH = (EPT + CHUNK - 1) // CHUNK
PAD_E = N_CH * CHUNK - EPT
N_PAD = N_CH * CHUNK
RPT = N_PAD // NS


def _make_seg_sum(with_counts):
  mesh = plsc.VectorSubcoreMesh(
      core_axis_name="c", subcore_axis_name="s", num_cores=NC, num_subcores=NS)
  out_type = [jax.ShapeDtypeStruct((NC, N_PAD, D), jnp.float32)]
  if with_counts:
    out_type.append(jax.ShapeDtypeStruct((NW, N_PAD), jnp.float32))
  scratch = [
      pltpu.VMEM((N_CH, CHUNK), jnp.int32),
      pltpu.VMEM((N_CH, CHUNK), jnp.int32),
      pltpu.VMEM((CHUNK, D), jnp.float32),
      pltpu.VMEM_SHARED((N_PAD, D), jnp.float32),
      pltpu.SemaphoreType.DMA,
  ]
  if with_counts:
    scratch.append(pltpu.VMEM((N_PAD,), jnp.float32))

  def body(table, srcp, dstp, zrows, *refs):
    if with_counts:
      acc_out, cnt_out, src_v, dst_v, rows_v, acc_sh, sem, cnt_v = refs
    else:
      acc_out, src_v, dst_v, rows_v, acc_sh, sem = refs

    cid = lax.axis_index("c")
    sid = lax.axis_index("s")
    wid = cid * NS + sid
    base = sid * RPT

    pltpu.sync_copy(srcp.at[wid], src_v)
    pltpu.sync_copy(dstp.at[wid], dst_v)
    pltpu.sync_copy(zrows.at[pl.ds(base, RPT)], acc_sh.at[pl.ds(base, RPT)])
    plsc.subcore_barrier()

    def edge_body(j, carry):
      pltpu.async_copy(table.at[src_v.at[j]], rows_v, sem).wait()
      pltpu.sync_copy(rows_v, acc_sh.at[dst_v.at[j]], add=True)
      return carry
    lax.fori_loop(0, N_CH, edge_body, 0)

    if with_counts:
      zero16 = jnp.zeros((L,), jnp.float32)
      ones16 = jnp.ones((L,), jnp.float32)

      def zero_body(i, carry):
        cnt_v[pl.ds(i * L, L)] = zero16
        return carry
      lax.fori_loop(0, N_PAD // L, zero_body, 0)

      def hist_body(i, carry):
        idx = dst_v[i // (CHUNK // L), pl.ds((i % (CHUNK // L)) * L, L)]
        plsc.addupdate_scatter(cnt_v, [idx], ones16)
        return carry
      lax.fori_loop(0, N_CH * (CHUNK // L), hist_body, 0)
      pltpu.sync_copy(cnt_v, cnt_out.at[wid])

    plsc.subcore_barrier()
    pltpu.sync_copy(acc_sh.at[pl.ds(base, RPT)],
                    acc_out.at[cid, pl.ds(base, RPT)])

  return pl.kernel(body, out_type=tuple(out_type), mesh=mesh,
                   scratch_types=scratch)


_seg_sum_counts = _make_seg_sum(True)
_seg_sum = _make_seg_sum(False)


def _tc_layer(p_ref, cnt_ref, x_ref, wl_ref, b_ref, wr_ref, o_ref, *,
              normalize):
  s = p_ref[0] + p_ref[1]
  c = jnp.sum(cnt_ref[...], axis=0)
  mean = s * (1.0 / jnp.maximum(c, 1.0))[:, None]
  o = (jnp.dot(mean, wl_ref[...], preferred_element_type=jnp.float32)
       + b_ref[...]
       + jnp.dot(x_ref[...], wr_ref[...], preferred_element_type=jnp.float32))
  if normalize:
    nrm = jnp.sqrt(jnp.sum(o * o, axis=1, keepdims=True))
    o = o / jnp.maximum(nrm, 1e-12)
  o_ref[...] = jnp.maximum(o, 0.0)


def _make_tc_layer(normalize):
  BM = 400
  grid = (N // BM,)
  return pl.pallas_call(
      functools.partial(_tc_layer, normalize=normalize),
      grid=grid,
      in_specs=[
          pl.BlockSpec((NC, BM, D), lambda i: (0, i, 0)),
          pl.BlockSpec((NW, BM), lambda i: (0, i)),
          pl.BlockSpec((BM, D), lambda i: (i, 0)),
          pl.BlockSpec((D, D), lambda i: (0, 0)),
          pl.BlockSpec((1, D), lambda i: (0, 0)),
          pl.BlockSpec((D, D), lambda i: (0, 0)),
      ],
      out_specs=pl.BlockSpec((BM, D), lambda i: (i, 0)),
      out_shape=jax.ShapeDtypeStruct((N, D), jnp.float32),
  )


_tc_layer1 = _make_tc_layer(False)
_tc_layer2 = _make_tc_layer(True)


def kernel(matrix_nodes_features, edge_index, W1l, b1, W1r, W2l, b2, W2r):
  x = matrix_nodes_features.astype(jnp.float32)
  ei = edge_index.astype(jnp.int32)
  src = ei[0].reshape(NW, EPT)
  dst = ei[1].reshape(NW, EPT)
  srcp = jnp.concatenate(
      [src, jnp.zeros((NW, PAD_E), jnp.int32)], axis=1).reshape(NW, N_CH, CHUNK)
  dstp = jnp.concatenate(
      [dst, jnp.full((NW, PAD_E), N, jnp.int32)], axis=1).reshape(NW, N_CH, CHUNK)
  zrows = jnp.zeros((N_PAD, D), jnp.float32)
  b1r = b1.reshape(1, D).astype(jnp.float32)
  b2r = b2.reshape(1, D).astype(jnp.float32)

  p1, cnt = _seg_sum_counts(x, srcp, dstp, zrows)
  h = _tc_layer1(p1, cnt, x, W1l.astype(jnp.float32), b1r,
                 W1r.astype(jnp.float32))
  p2 = _seg_sum(h, srcp, dstp, zrows)
  out = _tc_layer2(p2, cnt, h, W2l.astype(jnp.float32), b2r,
                   W2r.astype(jnp.float32))
  return out


# speedup vs baseline: 3.8633x; 3.8633x over previous
import jax
import jax.numpy as jnp
from jax import lax
from jax.experimental import pallas as pl
from jax.experimental.pallas import tpu as pltpu
from jax.experimental.pallas import tpu_sc as plsc

N = 10000
E = 320000
D = 128

NC = 2
NS = 16
NW = NC * NS
L = 16

CHUNK = 128
EPT = E // NW
N_CH = (EPT + CHUNK - 1) // CHUNK
PAD_E = N_CH * CHUNK - EPT
N_PAD = N_CH * CHUNK
RPT = N_PAD // NS

_MESH = dict(core_axis_name="c", subcore_axis_name="s",
             num_cores=NC, num_subcores=NS)
_SLAB = [(o, min(CHUNK, RPT - o)) for o in range(0, RPT, CHUNK)]


def _fill(ref, value, rows):
  v16 = jnp.full((L,), value, jnp.float32)

  def fb(i, carry):
    ref[i // (D // L), pl.ds((i % (D // L)) * L, L)] = v16
    return carry
  lax.fori_loop(0, rows * (D // L), fb, 0)


def _make_seg_sum():
  def body(table, srcp, dstp, acc_out, src_v, dst_v, rows_v, acc_sh, sem):
    cid = lax.axis_index("c")
    sid = lax.axis_index("s")
    wid = cid * NS + sid
    base = sid * RPT

    _fill(rows_v, 0.0, CHUNK)
    for off, sz in _SLAB:
      pltpu.sync_copy(rows_v.at[pl.ds(0, sz)],
                      acc_sh.at[pl.ds(base + off, sz)])
    plsc.subcore_barrier()

    def edge_body(j, carry):
      pltpu.sync_copy(srcp.at[wid, pl.ds(j, 1)], src_v)
      pltpu.sync_copy(dstp.at[wid, pl.ds(j, 1)], dst_v)
      pltpu.async_copy(table.at[src_v.at[0]], rows_v, sem).wait()
      pltpu.sync_copy(rows_v, acc_sh.at[dst_v.at[0]], add=True)
      return carry
    lax.fori_loop(0, N_CH, edge_body, 0)

    plsc.subcore_barrier()
    for off, sz in _SLAB:
      pltpu.sync_copy(acc_sh.at[pl.ds(base + off, sz)],
                      rows_v.at[pl.ds(0, sz)])
      pltpu.sync_copy(rows_v.at[pl.ds(0, sz)],
                      acc_out.at[cid, pl.ds(base + off, sz)])

  return pl.kernel(
      body,
      out_type=jax.ShapeDtypeStruct((NC, N_PAD, D), jnp.float32),
      mesh=plsc.VectorSubcoreMesh(**_MESH),
      scratch_types=[
          pltpu.VMEM((1, CHUNK), jnp.int32),
          pltpu.VMEM((1, CHUNK), jnp.int32),
          pltpu.VMEM((CHUNK, D), jnp.float32),
          pltpu.VMEM_SHARED((N_PAD, D), jnp.float32),
          pltpu.SemaphoreType.DMA,
      ])


def _make_counts():
  def body(dstp, cnt_out, dst_v, rows_v, cnt_sh):
    cid = lax.axis_index("c")
    sid = lax.axis_index("s")
    wid = cid * NS + sid
    base = sid * RPT

    _fill(rows_v, 0.0, CHUNK)
    for off, sz in _SLAB:
      pltpu.sync_copy(rows_v.at[pl.ds(0, sz)],
                      cnt_sh.at[pl.ds(base + off, sz)])
    _fill(rows_v, 1.0, CHUNK)
    plsc.subcore_barrier()

    def edge_body(j, carry):
      pltpu.sync_copy(dstp.at[wid, pl.ds(j, 1)], dst_v)
      pltpu.sync_copy(rows_v, cnt_sh.at[dst_v.at[0]], add=True)
      return carry
    lax.fori_loop(0, N_CH, edge_body, 0)

    plsc.subcore_barrier()
    for off, sz in _SLAB:
      pltpu.sync_copy(cnt_sh.at[pl.ds(base + off, sz)],
                      rows_v.at[pl.ds(0, sz)])
      pltpu.sync_copy(rows_v.at[pl.ds(0, sz)],
                      cnt_out.at[cid, pl.ds(base + off, sz)])

  return pl.kernel(
      body,
      out_type=jax.ShapeDtypeStruct((NC, N_PAD, D), jnp.float32),
      mesh=plsc.VectorSubcoreMesh(**_MESH),
      scratch_types=[
          pltpu.VMEM((1, CHUNK), jnp.int32),
          pltpu.VMEM((CHUNK, D), jnp.float32),
          pltpu.VMEM_SHARED((N_PAD, D), jnp.float32),
      ])


_seg_sum = _make_seg_sum()
_counts = _make_counts()


def _tc_layer1(p_ref, cnt_ref, x_ref, wl_ref, b_ref, wr_ref, o_ref, inv_ref):
  s = p_ref[0] + p_ref[1]
  c = cnt_ref[0, :, 0] + cnt_ref[1, :, 0]
  inv = 1.0 / jnp.maximum(c, 1.0)
  mean = s * inv[:, None]
  o = (jnp.dot(mean, wl_ref[...], preferred_element_type=jnp.float32)
       + b_ref[...]
       + jnp.dot(x_ref[...], wr_ref[...], preferred_element_type=jnp.float32))
  o_ref[...] = jnp.maximum(o, 0.0)
  inv_ref[...] = jnp.broadcast_to(inv[:, None], inv_ref.shape)


def _tc_layer2(p_ref, inv_ref, x_ref, wl_ref, b_ref, wr_ref, o_ref):
  s = p_ref[0] + p_ref[1]
  mean = s * inv_ref[:, :1]
  o = (jnp.dot(mean, wl_ref[...], preferred_element_type=jnp.float32)
       + b_ref[...]
       + jnp.dot(x_ref[...], wr_ref[...], preferred_element_type=jnp.float32))
  nrm = jnp.sqrt(jnp.sum(o * o, axis=1, keepdims=True))
  o = o / jnp.maximum(nrm, 1e-12)
  o_ref[...] = jnp.maximum(o, 0.0)


BM = 128
_GRID = (N_PAD // BM,)

_tc1 = pl.pallas_call(
    _tc_layer1,
    grid=_GRID,
    in_specs=[
        pl.BlockSpec((NC, BM, D), lambda i: (0, i, 0)),
        pl.BlockSpec((NC, BM, D), lambda i: (0, i, 0)),
        pl.BlockSpec((BM, D), lambda i: (i, 0)),
        pl.BlockSpec((D, D), lambda i: (0, 0)),
        pl.BlockSpec((1, D), lambda i: (0, 0)),
        pl.BlockSpec((D, D), lambda i: (0, 0)),
    ],
    out_specs=(pl.BlockSpec((BM, D), lambda i: (i, 0)),
               pl.BlockSpec((BM, 8), lambda i: (i, 0))),
    out_shape=(jax.ShapeDtypeStruct((N_PAD, D), jnp.float32),
               jax.ShapeDtypeStruct((N_PAD, 8), jnp.float32)),
)

_tc2 = pl.pallas_call(
    _tc_layer2,
    grid=_GRID,
    in_specs=[
        pl.BlockSpec((NC, BM, D), lambda i: (0, i, 0)),
        pl.BlockSpec((BM, 8), lambda i: (i, 0)),
        pl.BlockSpec((BM, D), lambda i: (i, 0)),
        pl.BlockSpec((D, D), lambda i: (0, 0)),
        pl.BlockSpec((1, D), lambda i: (0, 0)),
        pl.BlockSpec((D, D), lambda i: (0, 0)),
    ],
    out_specs=pl.BlockSpec((BM, D), lambda i: (i, 0)),
    out_shape=jax.ShapeDtypeStruct((N_PAD, D), jnp.float32),
)


def kernel(matrix_nodes_features, edge_index, W1l, b1, W1r, W2l, b2, W2r):
  x = matrix_nodes_features.astype(jnp.float32)
  ei = edge_index.astype(jnp.int32)
  src = ei[0].reshape(NW, EPT)
  dst = ei[1].reshape(NW, EPT)
  srcp = jnp.concatenate(
      [src, jnp.zeros((NW, PAD_E), jnp.int32)], axis=1).reshape(NW, N_CH, CHUNK)
  dstp = jnp.concatenate(
      [dst, jnp.full((NW, PAD_E), N, jnp.int32)], axis=1).reshape(NW, N_CH, CHUNK)
  xp = jnp.concatenate([x, jnp.zeros((N_PAD - N, D), jnp.float32)], axis=0)
  b1r = b1.reshape(1, D).astype(jnp.float32)
  b2r = b2.reshape(1, D).astype(jnp.float32)

  cnt = _counts(dstp)
  p1 = _seg_sum(xp, srcp, dstp)
  h, inv8 = _tc1(p1, cnt, xp, W1l.astype(jnp.float32), b1r,
                 W1r.astype(jnp.float32))
  p2 = _seg_sum(h, srcp, dstp)
  out = _tc2(p2, inv8, h, W2l.astype(jnp.float32), b2r,
             W2r.astype(jnp.float32))
  return out[:N]
